# baseline (speedup 1.0000x reference)
import functools

import jax
import jax.numpy as jnp
from jax import lax
from jax.experimental import pallas as pl
from jax.experimental.pallas import tpu as pltpu
from jax.experimental.pallas import tpu_sc as plsc

_POOL = 1000
_POOL_PAD = 1024
_EMBED = 128
_LENGTH = 8
_TOPK = 4
_BATCH = 1024

_NHALF = 2
_HB = _BATCH // _NHALF
_BR = 256
_GRID = _HB // _BR

_NC = 2
_NS = 16
_NW = _NC * _NS
_BPW = (_HB * _TOPK) // _NW
_CH = 32
_NCHUNK = _BPW // _CH


def _topk_body(q_ref, keys_ref, idx_ref, loss_ref):
    q = q_ref[...]
    keys = keys_ref[...]
    q2 = jnp.sum(q * q, axis=1, keepdims=True)
    k2 = jnp.sum(keys * keys, axis=1)[None, :]
    qk = lax.dot_general(q, keys, (((1,), (1,)), ((), ())),
                         preferred_element_type=jnp.float32)
    d2 = jnp.maximum(q2 + k2 - 2.0 * qk, 0.0)
    col = lax.broadcasted_iota(jnp.int32, d2.shape, 1)
    big = jnp.float32(1e30)
    d2 = jnp.where(col < _POOL, d2, big)

    total = jnp.float32(0.0)
    picks = []
    for _ in range(_TOPK):
        m = jnp.min(d2, axis=1, keepdims=True)
        am = jnp.min(jnp.where(d2 == m, col, jnp.int32(2**30)),
                     axis=1, keepdims=True)
        picks.append(am)
        total = total + jnp.sum(jnp.sqrt(m))
        d2 = jnp.where(col == am, big, d2)
    cat = jnp.concatenate(picks, axis=1)
    catf = cat.astype(jnp.float32)
    r_lane = lax.broadcasted_iota(jnp.int32, (_TOPK, 128), 1)
    r_row = lax.broadcasted_iota(jnp.int32, (_TOPK, 128), 0)
    rep = ((r_lane & 3) == r_row).astype(jnp.float32)
    a1 = lax.dot_general(catf, rep, (((1,), (0,)), ((), ())),
                         preferred_element_type=jnp.float32)
    lane = lax.broadcasted_iota(jnp.int32, (_BR, 128), 1)
    row = lax.broadcasted_iota(jnp.int32, (_BR, 128), 0)
    a2 = jnp.where((lane >> 2) == (row & 31), a1, 0.0)
    nrow = _BR * _TOPK // 128
    l_lane = lax.broadcasted_iota(jnp.int32, (nrow, _BR), 1)
    l_row = lax.broadcasted_iota(jnp.int32, (nrow, _BR), 0)
    lsum = ((l_lane >> 5) == l_row).astype(jnp.float32)
    m = lax.dot_general(lsum, a2, (((1,), (0,)), ((), ())),
                        preferred_element_type=jnp.float32)
    idx_ref[...] = m.astype(jnp.int32)
    loss_ref[...] = jnp.broadcast_to(total, (1, 1, 128))


def _topk_call(query, keys_pad):
    return pl.pallas_call(
        _topk_body,
        grid=(_GRID,),
        in_specs=[
            pl.BlockSpec((_BR, _EMBED), lambda i: (i, 0)),
            pl.BlockSpec((_POOL_PAD, _EMBED), lambda i: (0, 0)),
        ],
        out_specs=[
            pl.BlockSpec((_BR * _TOPK // 128, 128), lambda i: (i, 0)),
            pl.BlockSpec((1, 1, 128), lambda i: (i, 0, 0)),
        ],
        out_shape=[
            jax.ShapeDtypeStruct((_HB * _TOPK // 128, 128), jnp.int32),
            jax.ShapeDtypeStruct((_GRID, 1, 128), jnp.float32),
        ],
    )(query, keys_pad)


def _make_gather(half):
    base_half = half * _HB * _TOPK

    def _gather_body(table_hbm, idx_hbm, out_hbm, idx_v, rows_v, sem0, sem1):
        wid = lax.axis_index("s") * _NC + lax.axis_index("c")
        base = base_half + wid * _BPW
        per_row = 128 // _BPW
        pltpu.sync_copy(
            idx_hbm.at[wid // per_row,
                       pl.ds((wid % per_row) * _BPW, _BPW)],
            idx_v)
        sems = (sem0, sem1)
        cps = [None, None]
        for c in range(_NCHUNK):
            b = c % 2
            if cps[b] is not None:
                cps[b].wait()
                pltpu.sync_copy(rows_v.at[b],
                                out_hbm.at[pl.ds(base + (c - 2) * _CH, _CH)])
            cps[b] = pltpu.async_copy(
                table_hbm.at[idx_v.at[pl.ds(c * _CH, _CH)]],
                rows_v.at[b], sems[b])
        for c in range(_NCHUNK - 2, _NCHUNK):
            b = c % 2
            cps[b].wait()
            pltpu.sync_copy(rows_v.at[b],
                            out_hbm.at[pl.ds(base + c * _CH, _CH)])

    mesh = plsc.VectorSubcoreMesh(core_axis_name="c", subcore_axis_name="s")
    return pl.kernel(
        _gather_body,
        out_type=(),
        mesh=mesh,
        scratch_types=[
            pltpu.VMEM((_BPW,), jnp.int32),
            pltpu.VMEM((2, _CH, _LENGTH, _EMBED), jnp.float32),
            pltpu.SemaphoreType.DMA,
            pltpu.SemaphoreType.DMA,
        ],
        name=f"gather_half{half}",
    )


_gather_half = tuple(_make_gather(h) for h in range(_NHALF))


@jax.jit
def kernel(query, prompt_keys, prompt_values):
    keys_pad = jnp.pad(prompt_keys, ((0, _POOL_PAD - _POOL), (0, 0)))
    out_ref = jax.new_ref(
        lax.empty((_BATCH * _TOPK, _LENGTH, _EMBED), jnp.float32))
    loss = jnp.float32(0.0)
    for h in range(_NHALF):
        qh = lax.slice_in_dim(query, h * _HB, (h + 1) * _HB, axis=0)
        idx2, loss_parts = _topk_call(qh, keys_pad)
        loss = loss + jnp.sum(loss_parts[:, 0, 0])
        _gather_half[h](prompt_values, idx2, out_ref)
    rows = out_ref[...]
    quantized = rows.reshape(_BATCH, _TOPK, _LENGTH, _EMBED)
    return (quantized, loss / _BATCH)


# speedup vs baseline: 1.1340x; 1.1340x over previous
import functools

import jax
import jax.numpy as jnp
from jax import lax
from jax.experimental import pallas as pl
from jax.experimental.pallas import tpu as pltpu
from jax.experimental.pallas import tpu_sc as plsc

_POOL = 1000
_EMBED = 128
_LENGTH = 8
_TOPK = 4
_BATCH = 1024

_BR = 256
_GRID = _BATCH // _BR

_NC = 2
_NS = 16
_NW = _NC * _NS
_BPW = (_BATCH * _TOPK) // _NW
_CH = 32
_NCHUNK = _BPW // _CH


def _topk_body(q_ref, keys_ref, idx_ref, loss_ref):
    q = q_ref[...]
    keys = keys_ref[...]
    q2 = jnp.sum(q * q, axis=1, keepdims=True)
    k2 = jnp.sum(keys * keys, axis=1)[None, :]
    qk = lax.dot_general(q, keys, (((1,), (1,)), ((), ())),
                         preferred_element_type=jnp.float32)
    d2 = jnp.maximum(q2 + k2 - 2.0 * qk, 0.0)
    colf = lax.broadcasted_iota(jnp.int32, d2.shape, 1).astype(jnp.float32)
    big = jnp.float32(1e30)

    total = jnp.float32(0.0)
    picks = []
    for _ in range(_TOPK):
        m = jnp.min(d2, axis=1, keepdims=True)
        am = jnp.min(jnp.where(d2 == m, colf, big),
                     axis=1, keepdims=True)
        picks.append(am)
        total = total + jnp.sum(jnp.sqrt(m))
        d2 = jnp.where(colf == am, big, d2)
    cat = jnp.concatenate(picks, axis=1)
    r_lane = lax.broadcasted_iota(jnp.int32, (_TOPK, 128), 1)
    r_row = lax.broadcasted_iota(jnp.int32, (_TOPK, 128), 0)
    rep = ((r_lane & 3) == r_row).astype(jnp.float32)
    a1 = lax.dot_general(cat, rep, (((1,), (0,)), ((), ())),
                         preferred_element_type=jnp.float32)
    lane = lax.broadcasted_iota(jnp.int32, (_BR, 128), 1)
    row = lax.broadcasted_iota(jnp.int32, (_BR, 128), 0)
    a2 = jnp.where((lane >> 2) == (row & 31), a1, 0.0)
    nrow = _BR * _TOPK // 128
    l_lane = lax.broadcasted_iota(jnp.int32, (nrow, _BR), 1)
    l_row = lax.broadcasted_iota(jnp.int32, (nrow, _BR), 0)
    lsum = ((l_lane >> 5) == l_row).astype(jnp.float32)
    m2 = lax.dot_general(lsum, a2, (((1,), (0,)), ((), ())),
                         preferred_element_type=jnp.float32)
    idx_ref[...] = m2.astype(jnp.int32)
    loss_ref[...] = jnp.broadcast_to(total, (1, 1, 128))


def _topk_call(query, prompt_keys):
    return pl.pallas_call(
        _topk_body,
        grid=(_GRID,),
        in_specs=[
            pl.BlockSpec((_BR, _EMBED), lambda i: (i, 0)),
            pl.BlockSpec((_POOL, _EMBED), lambda i: (0, 0)),
        ],
        out_specs=[
            pl.BlockSpec((_BR * _TOPK // 128, 128), lambda i: (i, 0)),
            pl.BlockSpec((1, 1, 128), lambda i: (i, 0, 0)),
        ],
        out_shape=[
            jax.ShapeDtypeStruct((_BATCH * _TOPK // 128, 128), jnp.int32),
            jax.ShapeDtypeStruct((_GRID, 1, 128), jnp.float32),
        ],
    )(query, prompt_keys)


def _gather_body(table_hbm, idx_hbm, out_hbm, idx_v, rows_v, sem0, sem1):
    wid = lax.axis_index("s") * _NC + lax.axis_index("c")
    base = wid * _BPW
    pltpu.sync_copy(idx_hbm.at[wid], idx_v)
    sems = (sem0, sem1)
    cps = [None, None]
    for c in range(_NCHUNK):
        b = c % 2
        if cps[b] is not None:
            cps[b].wait()
            pltpu.sync_copy(rows_v.at[b],
                            out_hbm.at[pl.ds(base + (c - 2) * _CH, _CH)])
        cps[b] = pltpu.async_copy(
            table_hbm.at[idx_v.at[pl.ds(c * _CH, _CH)]],
            rows_v.at[b], sems[b])
    for c in range(_NCHUNK - 2, _NCHUNK):
        b = c % 2
        cps[b].wait()
        pltpu.sync_copy(rows_v.at[b],
                        out_hbm.at[pl.ds(base + c * _CH, _CH)])


def _gather_call(table, idx2):
    mesh = plsc.VectorSubcoreMesh(core_axis_name="c", subcore_axis_name="s")
    return pl.kernel(
        _gather_body,
        out_type=jax.ShapeDtypeStruct((_BATCH * _TOPK, _LENGTH, _EMBED),
                                      jnp.float32),
        mesh=mesh,
        scratch_types=[
            pltpu.VMEM((_BPW,), jnp.int32),
            pltpu.VMEM((2, _CH, _LENGTH, _EMBED), jnp.float32),
            pltpu.SemaphoreType.DMA,
            pltpu.SemaphoreType.DMA,
        ],
    )(table, idx2)


@jax.jit
def kernel(query, prompt_keys, prompt_values):
    idx2, loss_parts = _topk_call(query, prompt_keys)
    key_loss = jnp.sum(loss_parts[:, 0, 0]) / _BATCH
    rows = _gather_call(prompt_values, idx2)
    quantized = rows.reshape(_BATCH, _TOPK, _LENGTH, _EMBED)
    return (quantized, key_loss)


# speedup vs baseline: 1.1542x; 1.0179x over previous
import functools

import jax
import jax.numpy as jnp
from jax import lax
from jax.experimental import pallas as pl
from jax.experimental.pallas import tpu as pltpu
from jax.experimental.pallas import tpu_sc as plsc

_POOL = 1000
_EMBED = 128
_LENGTH = 8
_TOPK = 4
_BATCH = 1024

_BR = 512
_GRID = _BATCH // _BR

_NC = 2
_NS = 16
_NW = _NC * _NS
_BPW = (_BATCH * _TOPK) // _NW
_CH = 32
_NCHUNK = _BPW // _CH


def _topk_body(q_ref, keys_ref, idx_ref, loss_ref):
    q = q_ref[...]
    keys = keys_ref[...]
    q2 = jnp.sum(q * q, axis=1, keepdims=True)
    k2 = jnp.sum(keys * keys, axis=1)[None, :]
    qk = lax.dot_general(q, keys, (((1,), (1,)), ((), ())),
                         preferred_element_type=jnp.float32)
    d2 = jnp.maximum(q2 + k2 - 2.0 * qk, 0.0)
    colf = lax.broadcasted_iota(jnp.int32, d2.shape, 1).astype(jnp.float32)
    big = jnp.float32(1e30)

    total = jnp.float32(0.0)
    picks = []
    for _ in range(_TOPK):
        m = jnp.min(d2, axis=1, keepdims=True)
        am = jnp.min(jnp.where(d2 == m, colf, big),
                     axis=1, keepdims=True)
        picks.append(am)
        total = total + jnp.sum(jnp.sqrt(m))
        d2 = jnp.where(colf == am, big, d2)
    cat = jnp.concatenate(picks, axis=1)
    r_lane = lax.broadcasted_iota(jnp.int32, (_TOPK, 128), 1)
    r_row = lax.broadcasted_iota(jnp.int32, (_TOPK, 128), 0)
    rep = ((r_lane & 3) == r_row).astype(jnp.float32)
    a1 = lax.dot_general(cat, rep, (((1,), (0,)), ((), ())),
                         preferred_element_type=jnp.float32)
    lane = lax.broadcasted_iota(jnp.int32, (_BR, 128), 1)
    row = lax.broadcasted_iota(jnp.int32, (_BR, 128), 0)
    a2 = jnp.where((lane >> 2) == (row & 31), a1, 0.0)
    nrow = _BR * _TOPK // 128
    l_lane = lax.broadcasted_iota(jnp.int32, (nrow, _BR), 1)
    l_row = lax.broadcasted_iota(jnp.int32, (nrow, _BR), 0)
    lsum = ((l_lane >> 5) == l_row).astype(jnp.float32)
    m2 = lax.dot_general(lsum, a2, (((1,), (0,)), ((), ())),
                         preferred_element_type=jnp.float32)
    idx_ref[...] = m2.astype(jnp.int32)
    loss_ref[...] = jnp.broadcast_to(total, (1, 1, 128))


def _topk_call(query, prompt_keys):
    return pl.pallas_call(
        _topk_body,
        grid=(_GRID,),
        in_specs=[
            pl.BlockSpec((_BR, _EMBED), lambda i: (i, 0)),
            pl.BlockSpec((_POOL, _EMBED), lambda i: (0, 0)),
        ],
        out_specs=[
            pl.BlockSpec((_BR * _TOPK // 128, 128), lambda i: (i, 0)),
            pl.BlockSpec((1, 1, 128), lambda i: (i, 0, 0)),
        ],
        out_shape=[
            jax.ShapeDtypeStruct((_BATCH * _TOPK // 128, 128), jnp.int32),
            jax.ShapeDtypeStruct((_GRID, 1, 128), jnp.float32),
        ],
    )(query, prompt_keys)


def _gather_body(table_hbm, idx_hbm, out_hbm, idx_v, rows_v, sem0, sem1):
    wid = lax.axis_index("s") * _NC + lax.axis_index("c")
    base = wid * _BPW
    pltpu.sync_copy(idx_hbm.at[wid], idx_v)
    sems = (sem0, sem1)
    cps = [None, None]
    for c in range(_NCHUNK):
        b = c % 2
        if cps[b] is not None:
            cps[b].wait()
            pltpu.sync_copy(rows_v.at[b],
                            out_hbm.at[pl.ds(base + (c - 2) * _CH, _CH)])
        cps[b] = pltpu.async_copy(
            table_hbm.at[idx_v.at[pl.ds(c * _CH, _CH)]],
            rows_v.at[b], sems[b])
    for c in range(_NCHUNK - 2, _NCHUNK):
        b = c % 2
        cps[b].wait()
        pltpu.sync_copy(rows_v.at[b],
                        out_hbm.at[pl.ds(base + c * _CH, _CH)])


def _gather_call(table, idx2):
    mesh = plsc.VectorSubcoreMesh(core_axis_name="c", subcore_axis_name="s")
    return pl.kernel(
        _gather_body,
        out_type=jax.ShapeDtypeStruct((_BATCH * _TOPK, _LENGTH, _EMBED),
                                      jnp.float32),
        mesh=mesh,
        scratch_types=[
            pltpu.VMEM((_BPW,), jnp.int32),
            pltpu.VMEM((2, _CH, _LENGTH, _EMBED), jnp.float32),
            pltpu.SemaphoreType.DMA,
            pltpu.SemaphoreType.DMA,
        ],
    )(table, idx2)


@jax.jit
def kernel(query, prompt_keys, prompt_values):
    idx2, loss_parts = _topk_call(query, prompt_keys)
    key_loss = jnp.sum(loss_parts[:, 0, 0]) / _BATCH
    rows = _gather_call(prompt_values, idx2)
    quantized = rows.reshape(_BATCH, _TOPK, _LENGTH, _EMBED)
    return (quantized, key_loss)


# speedup vs baseline: 1.1679x; 1.0118x over previous
import functools

import jax
import jax.numpy as jnp
from jax import lax
from jax.experimental import pallas as pl
from jax.experimental.pallas import tpu as pltpu
from jax.experimental.pallas import tpu_sc as plsc

_POOL = 1000
_EMBED = 128
_LENGTH = 8
_TOPK = 4
_BATCH = 1024

_BR = 1024
_GRID = _BATCH // _BR

_NC = 2
_NS = 16
_NW = _NC * _NS
_BPW = (_BATCH * _TOPK) // _NW
_CH = 32
_NCHUNK = _BPW // _CH


def _topk_body(q_ref, keys_ref, idx_ref, loss_ref):
    q = q_ref[...]
    keys = keys_ref[...]
    q2 = jnp.sum(q * q, axis=1, keepdims=True)
    k2 = jnp.sum(keys * keys, axis=1)[None, :]
    qk = lax.dot_general(q, keys, (((1,), (1,)), ((), ())),
                         preferred_element_type=jnp.float32)
    d2 = jnp.maximum(q2 + k2 - 2.0 * qk, 0.0)
    colf = lax.broadcasted_iota(jnp.int32, d2.shape, 1).astype(jnp.float32)
    big = jnp.float32(1e30)

    total = jnp.float32(0.0)
    picks = []
    for _ in range(_TOPK):
        m = jnp.min(d2, axis=1, keepdims=True)
        am = jnp.min(jnp.where(d2 == m, colf, big),
                     axis=1, keepdims=True)
        picks.append(am)
        total = total + jnp.sum(jnp.sqrt(m))
        d2 = jnp.where(colf == am, big, d2)
    cat = jnp.concatenate(picks, axis=1)
    r_lane = lax.broadcasted_iota(jnp.int32, (_TOPK, 128), 1)
    r_row = lax.broadcasted_iota(jnp.int32, (_TOPK, 128), 0)
    rep = ((r_lane & 3) == r_row).astype(jnp.float32)
    a1 = lax.dot_general(cat, rep, (((1,), (0,)), ((), ())),
                         preferred_element_type=jnp.float32)
    lane = lax.broadcasted_iota(jnp.int32, (_BR, 128), 1)
    row = lax.broadcasted_iota(jnp.int32, (_BR, 128), 0)
    a2 = jnp.where((lane >> 2) == (row & 31), a1, 0.0)
    nrow = _BR * _TOPK // 128
    l_lane = lax.broadcasted_iota(jnp.int32, (nrow, _BR), 1)
    l_row = lax.broadcasted_iota(jnp.int32, (nrow, _BR), 0)
    lsum = ((l_lane >> 5) == l_row).astype(jnp.float32)
    m2 = lax.dot_general(lsum, a2, (((1,), (0,)), ((), ())),
                         preferred_element_type=jnp.float32)
    idx_ref[...] = m2.astype(jnp.int32)
    loss_ref[...] = jnp.broadcast_to(total, (1, 1, 128))


def _topk_call(query, prompt_keys):
    return pl.pallas_call(
        _topk_body,
        grid=(_GRID,),
        in_specs=[
            pl.BlockSpec((_BR, _EMBED), lambda i: (i, 0)),
            pl.BlockSpec((_POOL, _EMBED), lambda i: (0, 0)),
        ],
        out_specs=[
            pl.BlockSpec((_BR * _TOPK // 128, 128), lambda i: (i, 0)),
            pl.BlockSpec((1, 1, 128), lambda i: (i, 0, 0)),
        ],
        out_shape=[
            jax.ShapeDtypeStruct((_BATCH * _TOPK // 128, 128), jnp.int32),
            jax.ShapeDtypeStruct((_GRID, 1, 128), jnp.float32),
        ],
    )(query, prompt_keys)


def _gather_body(table_hbm, idx_hbm, out_hbm, idx_v, rows_v, sem0, sem1):
    wid = lax.axis_index("s") * _NC + lax.axis_index("c")
    base = wid * _BPW
    pltpu.sync_copy(idx_hbm.at[wid], idx_v)
    sems = (sem0, sem1)
    cps = [None, None]
    for c in range(_NCHUNK):
        b = c % 2
        if cps[b] is not None:
            cps[b].wait()
            pltpu.sync_copy(rows_v.at[b],
                            out_hbm.at[pl.ds(base + (c - 2) * _CH, _CH)])
        cps[b] = pltpu.async_copy(
            table_hbm.at[idx_v.at[pl.ds(c * _CH, _CH)]],
            rows_v.at[b], sems[b])
    for c in range(_NCHUNK - 2, _NCHUNK):
        b = c % 2
        cps[b].wait()
        pltpu.sync_copy(rows_v.at[b],
                        out_hbm.at[pl.ds(base + c * _CH, _CH)])


def _gather_call(table, idx2):
    mesh = plsc.VectorSubcoreMesh(core_axis_name="c", subcore_axis_name="s")
    return pl.kernel(
        _gather_body,
        out_type=jax.ShapeDtypeStruct((_BATCH * _TOPK, _LENGTH, _EMBED),
                                      jnp.float32),
        mesh=mesh,
        scratch_types=[
            pltpu.VMEM((_BPW,), jnp.int32),
            pltpu.VMEM((2, _CH, _LENGTH, _EMBED), jnp.float32),
            pltpu.SemaphoreType.DMA,
            pltpu.SemaphoreType.DMA,
        ],
    )(table, idx2)


@jax.jit
def kernel(query, prompt_keys, prompt_values):
    idx2, loss_parts = _topk_call(query, prompt_keys)
    key_loss = jnp.sum(loss_parts[:, 0, 0]) / _BATCH
    rows = _gather_call(prompt_values, idx2)
    quantized = rows.reshape(_BATCH, _TOPK, _LENGTH, _EMBED)
    return (quantized, key_loss)
